# Initial kernel scaffold; baseline (speedup 1.0000x reference)
import jax
import jax.numpy as jnp
from jax.experimental import pallas as pl


def kernel(positions, contraction, ln_scale, ln_bias, W1, W2, W_last, comp_w, edge_index, species, batch):
    raise NotImplementedError("write your pallas kernel here")


# speedup vs baseline: 6.6263x; 6.6263x over previous
import numpy as np
import jax
import jax.numpy as jnp
from jax.experimental import pallas as pl

_NSP = 4
_A = 4
_NMAX = 6
_LMAX = 3
_NLM = (_LMAX + 1) ** 2
_NSTRUCT = 16
_RCUT = 5.0
_HID = 256
_NFEAT = (_LMAX + 1) * _A * _A * _NMAX * _NMAX

_BE = 512
_BN = 256

def _build_perm():
    perm = np.zeros(_NFEAT, dtype=np.int32)
    for l in range(_LMAX + 1):
        for a in range(_A):
            for u in range(_NMAX):
                for b in range(_A):
                    for v in range(_NMAX):
                        f_mine = l * 576 + (a * _NMAX + u) * 24 + (b * _NMAX + v)
                        f_ref = l * 576 + a * 144 + b * 36 + u * 6 + v
                        perm[f_mine] = f_ref
    return perm

_PERM = _build_perm()


def _edge_kernel(psrc_ref, pdst_ref, out_ref):
    psrc = psrc_ref[...]
    pdst = pdst_ref[...]
    rij = pdst - psrc
    d2 = jnp.sum(rij * rij, axis=1, keepdims=True)
    d = jnp.sqrt(d2 + 1e-12)
    u = rij / d
    x = u[:, 0:1]; y = u[:, 1:2]; z = u[:, 2:3]
    x2 = x * x; y2 = y * y; z2 = z * z
    one = jnp.ones_like(x)
    Y = jnp.concatenate([
        0.28209479177387814 * one,
        0.4886025119029199 * y,
        0.4886025119029199 * z,
        0.4886025119029199 * x,
        1.0925484305920792 * x * y,
        1.0925484305920792 * y * z,
        0.31539156525252005 * (3.0 * z2 - 1.0),
        1.0925484305920792 * x * z,
        0.5462742152960396 * (x2 - y2),
        0.5900435899266435 * y * (3.0 * x2 - y2),
        2.890611442640554 * x * y * z,
        0.4570457994644658 * y * (5.0 * z2 - 1.0),
        0.3731763325901154 * z * (5.0 * z2 - 3.0),
        0.4570457994644658 * x * (5.0 * z2 - 1.0),
        1.445305721320277 * z * (x2 - y2),
        0.5900435899266435 * x * (x2 - 3.0 * y2),
    ], axis=1)
    fcut = jnp.where(d < _RCUT, 0.5 * (jnp.cos(np.pi * d / _RCUT) + 1.0), 0.0)
    mus = np.linspace(0.0, _RCUT, _NMAX)
    sigma = _RCUT / _NMAX
    inv2s2 = 1.0 / (2.0 * sigma * sigma)
    parts = [jnp.exp(-((d - float(mu)) ** 2) * inv2s2) * fcut * Y
             for mu in mus]
    out_ref[...] = jnp.concatenate(parts, axis=1)


def _node_kernel(spex_ref, contr_ref, scale_ref, bias_ref, w1_ref, w2_ref,
                 wl_ref, cw_ref, compw_ref, batch_ref, out_ref):
    i = pl.program_id(0)

    c = contr_ref[...]
    spex_a = []
    for a in range(_A):
        acc = c[0, a] * spex_ref[:, 0]
        for s in range(1, _NSP):
            acc = acc + c[s, a] * spex_ref[:, s]
        spex_a.append(acc)
    spex3 = jnp.concatenate(spex_a, axis=1)

    feats = []
    off = 0
    for l in range(_LMAX + 1):
        m = 2 * l + 1
        acc = None
        for k in range(m):
            v = spex3[:, :, off + k]
            w = v[:, :, None] * v[:, None, :]
            acc = w if acc is None else acc + w
        off += m
        acc = acc * (1.0 / np.sqrt(float(m)))
        feats.append(acc.reshape(acc.shape[0], 576))
    ps = jnp.concatenate(feats, axis=1)

    mean = jnp.mean(ps, axis=1, keepdims=True)
    var = jnp.mean(ps * ps, axis=1, keepdims=True) - mean * mean
    ln = (ps - mean) * jax.lax.rsqrt(var + 1e-5) * scale_ref[...] + bias_ref[...]

    t = jnp.dot(ln, w1_ref[...], preferred_element_type=jnp.float32)
    cw = cw_ref[...]
    esum = None
    for a in range(_A):
        h1 = jax.nn.silu(t[:, a * _HID:(a + 1) * _HID] * cw[:, a:a + 1])
        h2 = jax.nn.silu(jnp.dot(h1, w2_ref[a * _HID:(a + 1) * _HID, :],
                                 preferred_element_type=jnp.float32))
        e_a = jnp.dot(h2, wl_ref[a * _HID:(a + 1) * _HID, :],
                      preferred_element_type=jnp.float32)
        esum = e_a if esum is None else esum + e_a
    atomic = esum + compw_ref[...]

    rowid = i * _BN + jax.lax.broadcasted_iota(jnp.int32, (_BN, 1), 0)
    atomic = jnp.where(rowid < 10000, atomic, 0.0)
    seg = jax.lax.broadcasted_iota(jnp.int32, (_BN, _NSTRUCT), 1)
    onehot = jnp.where(batch_ref[...] == seg, 1.0, 0.0)
    contrib = jnp.sum(atomic * onehot, axis=0).reshape(1, _NSTRUCT)

    @pl.when(i == 0)
    def _():
        out_ref[...] = jnp.zeros_like(out_ref)
    out_ref[...] += contrib


def kernel(positions, contraction, ln_scale, ln_bias, W1, W2, W_last, comp_w,
           edge_index, species, batch):
    N = positions.shape[0]
    E = edge_index.shape[1]
    ei = edge_index.astype(jnp.int32)
    src = ei[0]
    dst = ei[1]
    species = species.astype(jnp.int32)
    batch = batch.astype(jnp.int32)

    psrc = positions[src]
    pdst = positions[dst]
    nbe = E // _BE
    cpair = pl.pallas_call(
        _edge_kernel,
        grid=(nbe,),
        in_specs=[
            pl.BlockSpec((_BE, 3), lambda i: (i, 0)),
            pl.BlockSpec((_BE, 3), lambda i: (i, 0)),
        ],
        out_specs=pl.BlockSpec((_BE, _NMAX * _NLM), lambda i: (i, 0)),
        out_shape=jax.ShapeDtypeStruct((E, _NMAX * _NLM), jnp.float32),
    )(psrc, pdst)

    idx = dst * _NSP + species[src]
    spex_sp = jnp.zeros((N * _NSP, _NMAX * _NLM), dtype=jnp.float32).at[idx].add(cpair)

    NP = ((N + _BN - 1) // _BN) * _BN
    pad = NP - N
    spex_sp = spex_sp.reshape(N, _NSP, _NMAX, _NLM)
    spex_sp = jnp.pad(spex_sp, ((0, pad), (0, 0), (0, 0), (0, 0)))

    cw = jnp.pad(contraction[species], ((0, pad), (0, 0)))
    compw_n = jnp.pad(comp_w[species], (0, pad)).reshape(NP, 1)
    batch_n = jnp.pad(batch, (0, pad)).reshape(NP, 1)

    perm = jnp.asarray(_PERM)
    scale_p = ln_scale[perm].reshape(1, _NFEAT)
    bias_p = ln_bias[perm].reshape(1, _NFEAT)
    w1_p = jnp.transpose(W1[:, perm, :], (1, 0, 2)).reshape(_NFEAT, _A * _HID)
    w2_p = W2.reshape(_A * _HID, _HID)
    wl_p = W_last.reshape(_A * _HID, 1)

    nbn = NP // _BN
    full = lambda i: (0, 0)
    total = pl.pallas_call(
        _node_kernel,
        grid=(nbn,),
        in_specs=[
            pl.BlockSpec((_BN, _NSP, _NMAX, _NLM), lambda i: (i, 0, 0, 0)),
            pl.BlockSpec((_NSP, _A), full),
            pl.BlockSpec((1, _NFEAT), full),
            pl.BlockSpec((1, _NFEAT), full),
            pl.BlockSpec((_NFEAT, _A * _HID), full),
            pl.BlockSpec((_A * _HID, _HID), full),
            pl.BlockSpec((_A * _HID, 1), full),
            pl.BlockSpec((_BN, _A), lambda i: (i, 0)),
            pl.BlockSpec((_BN, 1), lambda i: (i, 0)),
            pl.BlockSpec((_BN, 1), lambda i: (i, 0)),
        ],
        out_specs=pl.BlockSpec((1, _NSTRUCT), full),
        out_shape=jax.ShapeDtypeStruct((1, _NSTRUCT), jnp.float32),
    )(spex_sp, contraction, scale_p, bias_p, w1_p, w2_p, wl_p, cw, compw_n, batch_n)

    return total.reshape(_NSTRUCT) / np.sqrt(float(_A))


# speedup vs baseline: 7.6498x; 1.1545x over previous
import numpy as np
import jax
import jax.numpy as jnp
from jax.experimental import pallas as pl

_NSP = 4
_A = 4
_NMAX = 6
_LMAX = 3
_NLM = (_LMAX + 1) ** 2
_NSTRUCT = 16
_RCUT = 5.0
_HID = 256
_NFEAT = (_LMAX + 1) * _A * _A * _NMAX * _NMAX

_BE = 512
_BN = 256

def _build_perm():
    perm = np.zeros(_NFEAT, dtype=np.int32)
    for l in range(_LMAX + 1):
        for a in range(_A):
            for u in range(_NMAX):
                for b in range(_A):
                    for v in range(_NMAX):
                        f_mine = l * 576 + (a * _NMAX + u) * 24 + (b * _NMAX + v)
                        f_ref = l * 576 + a * 144 + b * 36 + u * 6 + v
                        perm[f_mine] = f_ref
    return perm

_PERM = _build_perm()


def _edge_kernel(psrc_ref, pdst_ref, out_ref):
    psrc = psrc_ref[...]
    pdst = pdst_ref[...]
    rij = pdst - psrc
    d2 = jnp.sum(rij * rij, axis=0, keepdims=True)
    d = jnp.sqrt(d2 + 1e-12)
    u = rij / d
    x = u[0:1]; y = u[1:2]; z = u[2:3]
    x2 = x * x; y2 = y * y; z2 = z * z
    one = jnp.ones_like(x)
    Y = jnp.concatenate([
        0.28209479177387814 * one,
        0.4886025119029199 * y,
        0.4886025119029199 * z,
        0.4886025119029199 * x,
        1.0925484305920792 * x * y,
        1.0925484305920792 * y * z,
        0.31539156525252005 * (3.0 * z2 - 1.0),
        1.0925484305920792 * x * z,
        0.5462742152960396 * (x2 - y2),
        0.5900435899266435 * y * (3.0 * x2 - y2),
        2.890611442640554 * x * y * z,
        0.4570457994644658 * y * (5.0 * z2 - 1.0),
        0.3731763325901154 * z * (5.0 * z2 - 3.0),
        0.4570457994644658 * x * (5.0 * z2 - 1.0),
        1.445305721320277 * z * (x2 - y2),
        0.5900435899266435 * x * (x2 - 3.0 * y2),
    ], axis=0)
    fcut = jnp.where(d < _RCUT, 0.5 * (jnp.cos(np.pi * d / _RCUT) + 1.0), 0.0)
    mus = np.linspace(0.0, _RCUT, _NMAX)
    sigma = _RCUT / _NMAX
    inv2s2 = 1.0 / (2.0 * sigma * sigma)
    parts = [jnp.exp(-((d - float(mu)) ** 2) * inv2s2) * fcut * Y
             for mu in mus]
    out_ref[...] = jnp.concatenate(parts, axis=0)


def _node_kernel(spex_ref, contr_ref, scale_ref, bias_ref, w1_ref, w2_ref,
                 wl_ref, cw_ref, compw_ref, batch_ref, out_ref):
    i = pl.program_id(0)

    c = contr_ref[...]
    spex_a = []
    for a in range(_A):
        acc = c[0, a] * spex_ref[:, 0]
        for s in range(1, _NSP):
            acc = acc + c[s, a] * spex_ref[:, s]
        spex_a.append(acc)
    spex3 = jnp.concatenate(spex_a, axis=1)

    feats = []
    off = 0
    for l in range(_LMAX + 1):
        m = 2 * l + 1
        acc = None
        for k in range(m):
            v = spex3[:, :, off + k]
            w = v[:, :, None] * v[:, None, :]
            acc = w if acc is None else acc + w
        off += m
        acc = acc * (1.0 / np.sqrt(float(m)))
        feats.append(acc.reshape(acc.shape[0], 576))
    ps = jnp.concatenate(feats, axis=1)

    mean = jnp.mean(ps, axis=1, keepdims=True)
    var = jnp.mean(ps * ps, axis=1, keepdims=True) - mean * mean
    ln = (ps - mean) * jax.lax.rsqrt(var + 1e-5) * scale_ref[...] + bias_ref[...]

    t = jnp.dot(ln, w1_ref[...], preferred_element_type=jnp.float32)
    cw = cw_ref[...]
    esum = None
    for a in range(_A):
        h1 = jax.nn.silu(t[:, a * _HID:(a + 1) * _HID] * cw[:, a:a + 1])
        h2 = jax.nn.silu(jnp.dot(h1, w2_ref[a * _HID:(a + 1) * _HID, :],
                                 preferred_element_type=jnp.float32))
        e_a = jnp.dot(h2, wl_ref[a * _HID:(a + 1) * _HID, :],
                      preferred_element_type=jnp.float32)
        esum = e_a if esum is None else esum + e_a
    atomic = esum + compw_ref[...]

    rowid = i * _BN + jax.lax.broadcasted_iota(jnp.int32, (_BN, 1), 0)
    atomic = jnp.where(rowid < 10000, atomic, 0.0)
    seg = jax.lax.broadcasted_iota(jnp.int32, (_BN, _NSTRUCT), 1)
    onehot = jnp.where(batch_ref[...] == seg, 1.0, 0.0)
    contrib = jnp.sum(atomic * onehot, axis=0).reshape(1, _NSTRUCT)

    @pl.when(i == 0)
    def _():
        out_ref[...] = jnp.zeros_like(out_ref)
    out_ref[...] += contrib


def kernel(positions, contraction, ln_scale, ln_bias, W1, W2, W_last, comp_w,
           edge_index, species, batch):
    N = positions.shape[0]
    E = edge_index.shape[1]
    ei = edge_index.astype(jnp.int32)
    src = ei[0]
    dst = ei[1]
    species = species.astype(jnp.int32)
    batch = batch.astype(jnp.int32)

    psrc = positions[src].T
    pdst = positions[dst].T
    nbe = E // _BE
    cpair_t = pl.pallas_call(
        _edge_kernel,
        grid=(nbe,),
        in_specs=[
            pl.BlockSpec((3, _BE), lambda i: (0, i)),
            pl.BlockSpec((3, _BE), lambda i: (0, i)),
        ],
        out_specs=pl.BlockSpec((_NMAX * _NLM, _BE), lambda i: (0, i)),
        out_shape=jax.ShapeDtypeStruct((_NMAX * _NLM, E), jnp.float32),
    )(psrc, pdst)
    cpair = cpair_t.T

    idx = dst * _NSP + species[src]
    spex_sp = jnp.zeros((N * _NSP, _NMAX * _NLM), dtype=jnp.float32).at[idx].add(cpair)

    NP = ((N + _BN - 1) // _BN) * _BN
    pad = NP - N
    spex_sp = spex_sp.reshape(N, _NSP, _NMAX, _NLM)
    spex_sp = jnp.pad(spex_sp, ((0, pad), (0, 0), (0, 0), (0, 0)))

    cw = jnp.pad(contraction[species], ((0, pad), (0, 0)))
    compw_n = jnp.pad(comp_w[species], (0, pad)).reshape(NP, 1)
    batch_n = jnp.pad(batch, (0, pad)).reshape(NP, 1)

    perm = jnp.asarray(_PERM)
    scale_p = ln_scale[perm].reshape(1, _NFEAT)
    bias_p = ln_bias[perm].reshape(1, _NFEAT)
    w1_p = jnp.transpose(W1[:, perm, :], (1, 0, 2)).reshape(_NFEAT, _A * _HID)
    w2_p = W2.reshape(_A * _HID, _HID)
    wl_p = W_last.reshape(_A * _HID, 1)

    nbn = NP // _BN
    full = lambda i: (0, 0)
    total = pl.pallas_call(
        _node_kernel,
        grid=(nbn,),
        in_specs=[
            pl.BlockSpec((_BN, _NSP, _NMAX, _NLM), lambda i: (i, 0, 0, 0)),
            pl.BlockSpec((_NSP, _A), full),
            pl.BlockSpec((1, _NFEAT), full),
            pl.BlockSpec((1, _NFEAT), full),
            pl.BlockSpec((_NFEAT, _A * _HID), full),
            pl.BlockSpec((_A * _HID, _HID), full),
            pl.BlockSpec((_A * _HID, 1), full),
            pl.BlockSpec((_BN, _A), lambda i: (i, 0)),
            pl.BlockSpec((_BN, 1), lambda i: (i, 0)),
            pl.BlockSpec((_BN, 1), lambda i: (i, 0)),
        ],
        out_specs=pl.BlockSpec((1, _NSTRUCT), full),
        out_shape=jax.ShapeDtypeStruct((1, _NSTRUCT), jnp.float32),
    )(spex_sp, contraction, scale_p, bias_p, w1_p, w2_p, wl_p, cw, compw_n, batch_n)

    return total.reshape(_NSTRUCT) / np.sqrt(float(_A))


# speedup vs baseline: 8.0001x; 1.0458x over previous
import numpy as np
import jax
import jax.numpy as jnp
from jax.experimental import pallas as pl

_NSP = 4
_A = 4
_NMAX = 6
_LMAX = 3
_NLM = (_LMAX + 1) ** 2
_NSTRUCT = 16
_RCUT = 5.0
_HID = 256
_NFEAT = (_LMAX + 1) * _A * _A * _NMAX * _NMAX

_BE = 512
_BN = 256

def _build_perm():
    perm = np.zeros(_NFEAT, dtype=np.int32)
    for l in range(_LMAX + 1):
        for a in range(_A):
            for u in range(_NMAX):
                for b in range(_A):
                    for v in range(_NMAX):
                        f_mine = l * 576 + (a * _NMAX + u) * 24 + (b * _NMAX + v)
                        f_ref = l * 576 + a * 144 + b * 36 + u * 6 + v
                        perm[f_mine] = f_ref
    return perm

_PERM = _build_perm()


def _edge_kernel(psrc_ref, pdst_ref, out_ref):
    psrc = psrc_ref[...]
    pdst = pdst_ref[...]
    rij = pdst - psrc
    d2 = jnp.sum(rij * rij, axis=0, keepdims=True)
    d = jnp.sqrt(d2 + 1e-12)
    u = rij / d
    x = u[0:1]; y = u[1:2]; z = u[2:3]
    x2 = x * x; y2 = y * y; z2 = z * z
    one = jnp.ones_like(x)
    Y = jnp.concatenate([
        0.28209479177387814 * one,
        0.4886025119029199 * y,
        0.4886025119029199 * z,
        0.4886025119029199 * x,
        1.0925484305920792 * x * y,
        1.0925484305920792 * y * z,
        0.31539156525252005 * (3.0 * z2 - 1.0),
        1.0925484305920792 * x * z,
        0.5462742152960396 * (x2 - y2),
        0.5900435899266435 * y * (3.0 * x2 - y2),
        2.890611442640554 * x * y * z,
        0.4570457994644658 * y * (5.0 * z2 - 1.0),
        0.3731763325901154 * z * (5.0 * z2 - 3.0),
        0.4570457994644658 * x * (5.0 * z2 - 1.0),
        1.445305721320277 * z * (x2 - y2),
        0.5900435899266435 * x * (x2 - 3.0 * y2),
    ], axis=0)
    fcut = jnp.where(d < _RCUT, 0.5 * (jnp.cos(np.pi * d / _RCUT) + 1.0), 0.0)
    mus = np.linspace(0.0, _RCUT, _NMAX)
    sigma = _RCUT / _NMAX
    inv2s2 = 1.0 / (2.0 * sigma * sigma)
    parts = [jnp.exp(-((d - float(mu)) ** 2) * inv2s2) * fcut * Y
             for mu in mus]
    out_ref[...] = jnp.concatenate(parts, axis=0)


def _node_kernel(spex_ref, contr_ref, scale_ref, bias_ref, w1_ref, w2_ref,
                 wl_ref, cw_ref, compw_ref, batch_ref, out_ref):
    i = pl.program_id(0)

    c = contr_ref[...]
    spex_a = []
    for a in range(_A):
        acc = c[0, a] * spex_ref[:, 0]
        for s in range(1, _NSP):
            acc = acc + c[s, a] * spex_ref[:, s]
        spex_a.append(acc)
    spex3 = jnp.concatenate(spex_a, axis=1)

    feats = []
    off = 0
    for l in range(_LMAX + 1):
        m = 2 * l + 1
        acc = None
        for k in range(m):
            v = spex3[:, :, off + k]
            w = v[:, :, None] * v[:, None, :]
            acc = w if acc is None else acc + w
        off += m
        acc = acc * (1.0 / np.sqrt(float(m)))
        feats.append(acc.reshape(acc.shape[0], 576))
    ps = jnp.concatenate(feats, axis=1)

    mean = jnp.mean(ps, axis=1, keepdims=True)
    var = jnp.mean(ps * ps, axis=1, keepdims=True) - mean * mean
    ln = (ps - mean) * jax.lax.rsqrt(var + 1e-5) * scale_ref[...] + bias_ref[...]

    t = jnp.dot(ln, w1_ref[...], preferred_element_type=jnp.float32)
    cw = cw_ref[...]
    esum = None
    for a in range(_A):
        h1 = jax.nn.silu(t[:, a * _HID:(a + 1) * _HID] * cw[:, a:a + 1])
        h2 = jax.nn.silu(jnp.dot(h1, w2_ref[a * _HID:(a + 1) * _HID, :],
                                 preferred_element_type=jnp.float32))
        e_a = jnp.dot(h2, wl_ref[a * _HID:(a + 1) * _HID, :],
                      preferred_element_type=jnp.float32)
        esum = e_a if esum is None else esum + e_a
    atomic = esum + compw_ref[...]

    rowid = i * _BN + jax.lax.broadcasted_iota(jnp.int32, (_BN, 1), 0)
    atomic = jnp.where(rowid < 10000, atomic, 0.0)
    seg = jax.lax.broadcasted_iota(jnp.int32, (_BN, _NSTRUCT), 1)
    onehot = jnp.where(batch_ref[...] == seg, 1.0, 0.0)
    contrib = jnp.sum(atomic * onehot, axis=0).reshape(1, _NSTRUCT)

    @pl.when(i == 0)
    def _():
        out_ref[...] = jnp.zeros_like(out_ref)
    out_ref[...] += contrib


def kernel(positions, contraction, ln_scale, ln_bias, W1, W2, W_last, comp_w,
           edge_index, species, batch):
    N = positions.shape[0]
    E = edge_index.shape[1]
    ei = edge_index.astype(jnp.int32)
    src = ei[0]
    dst = ei[1]
    species = species.astype(jnp.int32)
    batch = batch.astype(jnp.int32)

    psrc = positions[src].T
    pdst = positions[dst].T
    be = _BE
    for cand in (2560, 1280, 640):
        if E % cand == 0:
            be = cand
            break
    nbe = E // be
    cpair_t = pl.pallas_call(
        _edge_kernel,
        grid=(nbe,),
        in_specs=[
            pl.BlockSpec((3, be), lambda i: (0, i)),
            pl.BlockSpec((3, be), lambda i: (0, i)),
        ],
        out_specs=pl.BlockSpec((_NMAX * _NLM, be), lambda i: (0, i)),
        out_shape=jax.ShapeDtypeStruct((_NMAX * _NLM, E), jnp.float32),
    )(psrc, pdst)
    cpair = cpair_t.T

    idx = dst * _NSP + species[src]
    spex_sp = jnp.zeros((N * _NSP, _NMAX * _NLM), dtype=jnp.float32).at[idx].add(cpair)

    NP = ((N + _BN - 1) // _BN) * _BN
    pad = NP - N
    spex_sp = spex_sp.reshape(N, _NSP, _NMAX, _NLM)
    spex_sp = jnp.pad(spex_sp, ((0, pad), (0, 0), (0, 0), (0, 0)))

    cw = jnp.pad(contraction[species], ((0, pad), (0, 0)))
    compw_n = jnp.pad(comp_w[species], (0, pad)).reshape(NP, 1)
    batch_n = jnp.pad(batch, (0, pad)).reshape(NP, 1)

    perm = jnp.asarray(_PERM)
    scale_p = ln_scale[perm].reshape(1, _NFEAT)
    bias_p = ln_bias[perm].reshape(1, _NFEAT)
    w1_p = jnp.transpose(W1[:, perm, :], (1, 0, 2)).reshape(_NFEAT, _A * _HID)
    w2_p = W2.reshape(_A * _HID, _HID)
    wl_p = W_last.reshape(_A * _HID, 1)

    nbn = NP // _BN
    full = lambda i: (0, 0)
    total = pl.pallas_call(
        _node_kernel,
        grid=(nbn,),
        in_specs=[
            pl.BlockSpec((_BN, _NSP, _NMAX, _NLM), lambda i: (i, 0, 0, 0)),
            pl.BlockSpec((_NSP, _A), full),
            pl.BlockSpec((1, _NFEAT), full),
            pl.BlockSpec((1, _NFEAT), full),
            pl.BlockSpec((_NFEAT, _A * _HID), full),
            pl.BlockSpec((_A * _HID, _HID), full),
            pl.BlockSpec((_A * _HID, 1), full),
            pl.BlockSpec((_BN, _A), lambda i: (i, 0)),
            pl.BlockSpec((_BN, 1), lambda i: (i, 0)),
            pl.BlockSpec((_BN, 1), lambda i: (i, 0)),
        ],
        out_specs=pl.BlockSpec((1, _NSTRUCT), full),
        out_shape=jax.ShapeDtypeStruct((1, _NSTRUCT), jnp.float32),
    )(spex_sp, contraction, scale_p, bias_p, w1_p, w2_p, wl_p, cw, compw_n, batch_n)

    return total.reshape(_NSTRUCT) / np.sqrt(float(_A))
